# Initial kernel scaffold; baseline (speedup 1.0000x reference)
import jax
import jax.numpy as jnp
from jax.experimental import pallas as pl


def kernel(audio_ids, embed_table, W_out):
    raise NotImplementedError("write your pallas kernel here")


# speedup vs baseline: 1.9961x; 1.9961x over previous
import functools

import jax
import jax.numpy as jnp
from jax import lax
from jax.experimental import pallas as pl
from jax.experimental.pallas import tpu as pltpu
from jax.experimental.pallas import tpu_sc as plsc

GROUP = 5


def _sc_gather(table, idx_flat, n_rows, d, nw, k_rows):
    mesh = plsc.VectorSubcoreMesh(core_axis_name="c", subcore_axis_name="s")
    n_chunks = n_rows // k_rows
    base_chunks = n_chunks // nw
    extra = n_chunks - base_chunks * nw
    max_chunks = base_chunks + 1
    win = max_chunks * k_rows

    @functools.partial(
        pl.kernel,
        mesh=mesh,
        out_type=jax.ShapeDtypeStruct((n_rows, d), jnp.float32),
        scratch_types=[
            pltpu.VMEM((win,), jnp.int32),
            pltpu.VMEM((k_rows, d), jnp.float32),
            pltpu.SemaphoreType.DMA,
        ],
    )
    def gather_kernel(table_hbm, idx_hbm, out_hbm, idx_v, rows_v, sem):
        wid = lax.axis_index("s") * 2 + lax.axis_index("c")
        start = base_chunks * wid + jnp.minimum(wid, extra)
        my_chunks = base_chunks + jnp.where(wid < extra, 1, 0)
        pltpu.sync_copy(idx_hbm.at[pl.ds(start * k_rows, win)], idx_v)

        def body(i, carry):
            pltpu.async_copy(
                table_hbm.at[idx_v.at[pl.ds(i * k_rows, k_rows)]], rows_v, sem
            ).wait()
            pltpu.sync_copy(
                rows_v, out_hbm.at[pl.ds((start + i) * k_rows, k_rows)]
            )
            return carry

        lax.fori_loop(0, my_chunks, body, 0)

    return gather_kernel(table, idx_flat)


def _tc_pool_matmul(g3, w_bf16, ng, d, bm, bk):
    def body(a_ref, w_ref, o_ref):
        k = pl.program_id(1)
        s = a_ref[0] + a_ref[1] + a_ref[2] + a_ref[3] + a_ref[4]
        pooled = (s * (1.0 / GROUP)).astype(jnp.bfloat16)
        part = lax.dot_general(
            pooled,
            w_ref[...],
            (((1,), (1,)), ((), ())),
            preferred_element_type=jnp.float32,
        )

        @pl.when(k == 0)
        def _():
            o_ref[...] = part

        @pl.when(k != 0)
        def _():
            o_ref[...] += part

    return pl.pallas_call(
        body,
        grid=(ng // bm, d // bk),
        in_specs=[
            pl.BlockSpec((GROUP, bm, bk), lambda i, k: (0, i, k)),
            pl.BlockSpec((d, bk), lambda i, k: (0, k)),
        ],
        out_specs=pl.BlockSpec((bm, d), lambda i, k: (i, 0)),
        out_shape=jax.ShapeDtypeStruct((ng, d), jnp.float32),
    )(g3, w_bf16)


def kernel(audio_ids, embed_table, W_out):
    b, s = audio_ids.shape
    v, d = embed_table.shape
    ng = (b * s) // GROUP
    n_rows = b * s

    nw = 32
    k_rows = 16

    ids = audio_ids.reshape(-1).astype(jnp.int32)
    idx_perm = ids.reshape(ng, GROUP).T.reshape(-1)
    max_chunks = (n_rows // k_rows + nw - 1) // nw
    pad = nw * max_chunks * k_rows - n_rows
    idx_perm = jnp.concatenate([idx_perm, jnp.zeros((pad,), jnp.int32)])

    gathered = _sc_gather(embed_table, idx_perm, n_rows, d, nw, k_rows)
    g3 = gathered.reshape(GROUP, ng, d)

    out = _tc_pool_matmul(g3, W_out.astype(jnp.bfloat16), ng, d, bm=800, bk=512)
    return out.reshape(b, s // GROUP, d)
